# baseline (speedup 1.0000x reference)
import functools

import jax
import jax.numpy as jnp
from jax import lax
from jax.experimental import pallas as pl
from jax.experimental.pallas import tpu as pltpu
from jax.experimental.pallas import tpu_sc as plsc

N_NODES = 10000
N_EDGES = 320000
N_LABEL = 128
D = 128
DSUB = 8

NC = 2
NS = 16
NW = NC * NS
CHUNK = 200
SPLITS = (83200, 83200, 83200, 70400)
OFFS = (0, 83200, 166400, 249600)
GSPLIT = ((0, 104), (104, 96))


def _fire_gathers(h, idxa, idxb, bufa, bufb, sem):
    for off, ln in GSPLIT:
        pltpu.async_copy(h.at[idxa.at[pl.ds(off, ln)]],
                         bufa.at[pl.ds(off, ln)], sem)
        pltpu.async_copy(h.at[idxb.at[pl.ds(off, ln)]],
                         bufb.at[pl.ds(off, ln)], sem)


def _wait_gathers(h, idxa, idxb, bufa, bufb, sem):
    for off, ln in GSPLIT:
        pltpu.make_async_copy(h.at[idxa.at[pl.ds(off, ln)]],
                              bufa.at[pl.ds(off, ln)], sem).wait()
        pltpu.make_async_copy(h.at[idxb.at[pl.ds(off, ln)]],
                              bufb.at[pl.ds(off, ln)], sem).wait()


def _add_rows(bufa, bufb):
    def body(e2, carry):
        for r in range(2):
            e = e2 * 2 + r
            for c in range(DSUB):
                sl = pl.ds(c * 16, 16)
                bufa[e, sl] = bufa[e, sl] + bufb[e, sl]
        return carry
    lax.fori_loop(0, CHUNK // 2, body, 0)


def _sc_gather_body(do_label, n_edges, offset,
                    h, ne0, ne1, le0, le1, s2, rla, rlb,
                    idxa0, idxb0, idxa1, idxb1,
                    bufa0, bufb0, bufa1, bufb1, idxl,
                    gsem0, gsem1, wsem0, wsem1, lsem):
    epw = n_edges // NW
    nchunk = epw // CHUNK
    npair = (nchunk - 1) // 2
    c = lax.axis_index("c")
    s = lax.axis_index("s")
    wid = s * NC + c
    base0 = wid * epw

    if do_label:
        @pl.when(wid == 0)
        def _label():
            pltpu.sync_copy(le0, idxl)
            pltpu.async_copy(h.at[idxl], bufa0.at[pl.ds(0, N_LABEL)],
                             lsem).wait()
            pltpu.sync_copy(bufa0.at[pl.ds(0, N_LABEL)], rla)
            pltpu.sync_copy(le1, idxl)
            pltpu.async_copy(h.at[idxl], bufa0.at[pl.ds(0, N_LABEL)],
                             lsem).wait()
            pltpu.sync_copy(bufa0.at[pl.ds(0, N_LABEL)], rlb)

    def load_idx(j, idxa, idxb):
        base = pl.multiple_of(offset + base0 + j * CHUNK, 8)
        pltpu.sync_copy(ne0.at[pl.ds(base, CHUNK)], idxa)
        pltpu.sync_copy(ne1.at[pl.ds(base, CHUNK)], idxb)
        return base

    def process(j, idxa, idxb, bufa, bufb, gsem, wsem):
        _wait_gathers(h, idxa, idxb, bufa, bufb, gsem)
        _add_rows(bufa, bufb)
        b = pl.multiple_of(base0 + j * CHUNK, 8)
        pltpu.async_copy(bufa, s2.at[pl.ds(b, CHUNK)], wsem)

    load_idx(0, idxa0, idxb0)
    _fire_gathers(h, idxa0, idxb0, bufa0, bufb0, gsem0)

    def body(k, carry):
        j0 = 2 * k
        @pl.when(k > 0)
        def _():
            pltpu.make_async_copy(bufa1, s2.at[pl.ds(base0, CHUNK)],
                                  wsem1).wait()
        load_idx(j0 + 1, idxa1, idxb1)
        _fire_gathers(h, idxa1, idxb1, bufa1, bufb1, gsem1)
        process(j0, idxa0, idxb0, bufa0, bufb0, gsem0, wsem0)
        pltpu.make_async_copy(bufa0, s2.at[pl.ds(base0, CHUNK)],
                              wsem0).wait()
        load_idx(j0 + 2, idxa0, idxb0)
        _fire_gathers(h, idxa0, idxb0, bufa0, bufb0, gsem0)
        process(j0 + 1, idxa1, idxb1, bufa1, bufb1, gsem1, wsem1)
        return carry

    lax.fori_loop(0, npair, body, 0)

    process(nchunk - 1, idxa0, idxb0, bufa0, bufb0, gsem0, wsem0)
    pltpu.make_async_copy(bufa0, s2.at[pl.ds(base0, CHUNK)], wsem0).wait()
    pltpu.make_async_copy(bufa1, s2.at[pl.ds(base0, CHUNK)], wsem1).wait()


@functools.cache
def _make_sc_gather(do_label, n_edges, offset):
    return functools.partial(
        pl.kernel,
        out_type=[
            jax.ShapeDtypeStruct((n_edges, D), jnp.float32),
            jax.ShapeDtypeStruct((N_LABEL, D), jnp.float32),
            jax.ShapeDtypeStruct((N_LABEL, D), jnp.float32),
        ],
        mesh=plsc.VectorSubcoreMesh(core_axis_name="c", subcore_axis_name="s"),
        scratch_types=[
            pltpu.VMEM((CHUNK,), jnp.int32),
            pltpu.VMEM((CHUNK,), jnp.int32),
            pltpu.VMEM((CHUNK,), jnp.int32),
            pltpu.VMEM((CHUNK,), jnp.int32),
            pltpu.VMEM((CHUNK, D), jnp.float32),
            pltpu.VMEM((CHUNK, D), jnp.float32),
            pltpu.VMEM((CHUNK, D), jnp.float32),
            pltpu.VMEM((CHUNK, D), jnp.float32),
            pltpu.VMEM((N_LABEL,), jnp.int32),
            pltpu.SemaphoreType.DMA,
            pltpu.SemaphoreType.DMA,
            pltpu.SemaphoreType.DMA,
            pltpu.SemaphoreType.DMA,
            pltpu.SemaphoreType.DMA,
        ],
    )(functools.partial(_sc_gather_body, do_label, n_edges, offset))


B2 = 6400


def _tc_body(s2_ref, rla_ref, rlb_ref, le_ref,
             score_ref, idx_ref, pe_ref):
    en = s2_ref[...] * 0.5
    el = (rla_ref[...] + rlb_ref[...]) * 0.5
    sq_l = jnp.sum(el * el, axis=1, keepdims=True)
    dott = lax.dot_general(el, en, (((1,), (1,)), ((), ())),
                           preferred_element_type=jnp.float32)
    sq_nt = lax.dot_general(jnp.ones((1, D), jnp.float32), en * en,
                            (((1,), (1,)), ((), ())),
                            preferred_element_type=jnp.float32)
    d2t = jnp.maximum(sq_l + sq_nt - 2.0 * dott, 1e-12)
    mt = jnp.min(d2t, axis=0, keepdims=True)
    iot = lax.broadcasted_iota(jnp.int32, (N_LABEL, B2), 0)
    idxt = jnp.min(jnp.where(d2t == mt, iot, jnp.int32(N_LABEL)),
                   axis=0, keepdims=True)
    ohf = (iot == idxt).astype(jnp.float32)
    pe = lax.dot_general(le_ref[...].astype(jnp.float32), ohf,
                         (((1,), (0,)), ((), ())),
                         preferred_element_type=jnp.float32)
    score_ref[...] = -jnp.sqrt(mt)
    idx_ref[...] = idxt
    pe_ref[...] = pe.astype(jnp.int32)


@functools.cache
def _make_tc_math(n_edges):
    return pl.pallas_call(
        _tc_body,
        grid=(n_edges // B2,),
        in_specs=[
            pl.BlockSpec((B2, D), lambda i: (i, 0)),
            pl.BlockSpec((N_LABEL, D), lambda i: (0, 0)),
            pl.BlockSpec((N_LABEL, D), lambda i: (0, 0)),
            pl.BlockSpec((2, N_LABEL), lambda i: (0, 0)),
        ],
        out_specs=[
            pl.BlockSpec((1, B2), lambda i: (0, i)),
            pl.BlockSpec((1, B2), lambda i: (0, i)),
            pl.BlockSpec((2, B2), lambda i: (0, i)),
        ],
        out_shape=[
            jax.ShapeDtypeStruct((1, n_edges), jnp.float32),
            jax.ShapeDtypeStruct((1, n_edges), jnp.int32),
            jax.ShapeDtypeStruct((2, n_edges), jnp.int32),
        ],
    )


def kernel(h, node_edge, label_edge):
    le0, le1 = label_edge[0], label_edge[1]
    ne0, ne1 = node_edge[0], node_edge[1]
    parts = []
    rla = rlb = None
    for i, (sz, off) in enumerate(zip(SPLITS, OFFS)):
        s2_i, rla_i, rlb_i = _make_sc_gather(i == 0, sz, off)(
            h, ne0, ne1, le0, le1)
        if i == 0:
            rla, rlb = rla_i, rlb_i
        parts.append(s2_i)
    outs = [_make_tc_math(sz)(s2_i, rla, rlb, label_edge)
            for sz, s2_i in zip(SPLITS, parts)]
    n = node_edge.shape[1]
    return (jnp.concatenate([o[0] for o in outs], axis=1).reshape(n),
            jnp.concatenate([o[1] for o in outs], axis=1).reshape(n),
            jnp.concatenate([o[2] for o in outs], axis=1))


# speedup vs baseline: 1.1000x; 1.1000x over previous
import functools

import jax
import jax.numpy as jnp
from jax import lax
from jax.experimental import pallas as pl
from jax.experimental.pallas import tpu as pltpu
from jax.experimental.pallas import tpu_sc as plsc

N_NODES = 10000
N_EDGES = 320000
N_LABEL = 128
D = 128
DSUB = 8

NC = 2
NS = 16
NW = NC * NS
CHUNK = 200
SPLITS = (160000, 160000)
OFFS = (0, 160000)
GSPLIT = ((0, 104), (104, 96))


def _fire_gathers(h, idxa, idxb, bufa, bufb, sem):
    for off, ln in GSPLIT:
        pltpu.async_copy(h.at[idxa.at[pl.ds(off, ln)]],
                         bufa.at[pl.ds(off, ln)], sem)
        pltpu.async_copy(h.at[idxb.at[pl.ds(off, ln)]],
                         bufb.at[pl.ds(off, ln)], sem)


def _wait_gathers(h, idxa, idxb, bufa, bufb, sem):
    for off, ln in GSPLIT:
        pltpu.make_async_copy(h.at[idxa.at[pl.ds(off, ln)]],
                              bufa.at[pl.ds(off, ln)], sem).wait()
        pltpu.make_async_copy(h.at[idxb.at[pl.ds(off, ln)]],
                              bufb.at[pl.ds(off, ln)], sem).wait()


def _add_rows(bufa, bufb):
    def body(e2, carry):
        for r in range(2):
            e = e2 * 2 + r
            for c in range(DSUB):
                sl = pl.ds(c * 16, 16)
                bufa[e, sl] = bufa[e, sl] + bufb[e, sl]
        return carry
    lax.fori_loop(0, CHUNK // 2, body, 0)


def _sc_gather_body(do_label, n_edges, offset,
                    h, ne0, ne1, le0, le1, s2, rla, rlb,
                    idxa0, idxb0, idxa1, idxb1,
                    bufa0, bufb0, bufa1, bufb1, idxl,
                    gsem0, gsem1, wsem0, wsem1, lsem):
    epw = n_edges // NW
    nchunk = epw // CHUNK
    npair = (nchunk - 1) // 2
    c = lax.axis_index("c")
    s = lax.axis_index("s")
    wid = s * NC + c
    base0 = wid * epw

    if do_label:
        @pl.when(wid == 0)
        def _label():
            pltpu.sync_copy(le0, idxl)
            pltpu.async_copy(h.at[idxl], bufa0.at[pl.ds(0, N_LABEL)],
                             lsem).wait()
            pltpu.sync_copy(bufa0.at[pl.ds(0, N_LABEL)], rla)
            pltpu.sync_copy(le1, idxl)
            pltpu.async_copy(h.at[idxl], bufa0.at[pl.ds(0, N_LABEL)],
                             lsem).wait()
            pltpu.sync_copy(bufa0.at[pl.ds(0, N_LABEL)], rlb)

    def load_idx(j, idxa, idxb):
        base = pl.multiple_of(offset + base0 + j * CHUNK, 8)
        pltpu.sync_copy(ne0.at[pl.ds(base, CHUNK)], idxa)
        pltpu.sync_copy(ne1.at[pl.ds(base, CHUNK)], idxb)
        return base

    def process(j, idxa, idxb, bufa, bufb, gsem, wsem):
        _wait_gathers(h, idxa, idxb, bufa, bufb, gsem)
        _add_rows(bufa, bufb)
        b = pl.multiple_of(base0 + j * CHUNK, 8)
        pltpu.async_copy(bufa, s2.at[pl.ds(b, CHUNK)], wsem)

    load_idx(0, idxa0, idxb0)
    _fire_gathers(h, idxa0, idxb0, bufa0, bufb0, gsem0)

    def body(k, carry):
        j0 = 2 * k
        @pl.when(k > 0)
        def _():
            pltpu.make_async_copy(bufa1, s2.at[pl.ds(base0, CHUNK)],
                                  wsem1).wait()
        load_idx(j0 + 1, idxa1, idxb1)
        _fire_gathers(h, idxa1, idxb1, bufa1, bufb1, gsem1)
        process(j0, idxa0, idxb0, bufa0, bufb0, gsem0, wsem0)
        pltpu.make_async_copy(bufa0, s2.at[pl.ds(base0, CHUNK)],
                              wsem0).wait()
        load_idx(j0 + 2, idxa0, idxb0)
        _fire_gathers(h, idxa0, idxb0, bufa0, bufb0, gsem0)
        process(j0 + 1, idxa1, idxb1, bufa1, bufb1, gsem1, wsem1)
        return carry

    lax.fori_loop(0, npair, body, 0)

    process(nchunk - 1, idxa0, idxb0, bufa0, bufb0, gsem0, wsem0)
    pltpu.make_async_copy(bufa0, s2.at[pl.ds(base0, CHUNK)], wsem0).wait()
    pltpu.make_async_copy(bufa1, s2.at[pl.ds(base0, CHUNK)], wsem1).wait()


@functools.cache
def _make_sc_gather(do_label, n_edges, offset):
    return functools.partial(
        pl.kernel,
        out_type=[
            jax.ShapeDtypeStruct((n_edges, D), jnp.float32),
            jax.ShapeDtypeStruct((N_LABEL, D), jnp.float32),
            jax.ShapeDtypeStruct((N_LABEL, D), jnp.float32),
        ],
        mesh=plsc.VectorSubcoreMesh(core_axis_name="c", subcore_axis_name="s"),
        scratch_types=[
            pltpu.VMEM((CHUNK,), jnp.int32),
            pltpu.VMEM((CHUNK,), jnp.int32),
            pltpu.VMEM((CHUNK,), jnp.int32),
            pltpu.VMEM((CHUNK,), jnp.int32),
            pltpu.VMEM((CHUNK, D), jnp.float32),
            pltpu.VMEM((CHUNK, D), jnp.float32),
            pltpu.VMEM((CHUNK, D), jnp.float32),
            pltpu.VMEM((CHUNK, D), jnp.float32),
            pltpu.VMEM((N_LABEL,), jnp.int32),
            pltpu.SemaphoreType.DMA,
            pltpu.SemaphoreType.DMA,
            pltpu.SemaphoreType.DMA,
            pltpu.SemaphoreType.DMA,
            pltpu.SemaphoreType.DMA,
        ],
    )(functools.partial(_sc_gather_body, do_label, n_edges, offset))


B2 = 6400


def _tc_body(s2_ref, rla_ref, rlb_ref, le_ref,
             score_ref, idx_ref, pe_ref):
    en = s2_ref[...] * 0.5
    el = (rla_ref[...] + rlb_ref[...]) * 0.5
    sq_l = jnp.sum(el * el, axis=1, keepdims=True)
    dott = lax.dot_general(el, en, (((1,), (1,)), ((), ())),
                           preferred_element_type=jnp.float32)
    sq_nt = lax.dot_general(jnp.ones((1, D), jnp.float32), en * en,
                            (((1,), (1,)), ((), ())),
                            preferred_element_type=jnp.float32)
    d2t = jnp.maximum(sq_l + sq_nt - 2.0 * dott, 1e-12)
    mt = jnp.min(d2t, axis=0, keepdims=True)
    iot = lax.broadcasted_iota(jnp.int32, (N_LABEL, B2), 0)
    idxt = jnp.min(jnp.where(d2t == mt, iot, jnp.int32(N_LABEL)),
                   axis=0, keepdims=True)
    ohf = (iot == idxt).astype(jnp.float32)
    pe = lax.dot_general(le_ref[...].astype(jnp.float32), ohf,
                         (((1,), (0,)), ((), ())),
                         preferred_element_type=jnp.float32)
    score_ref[...] = -jnp.sqrt(mt)
    idx_ref[...] = idxt
    pe_ref[...] = pe.astype(jnp.int32)


@functools.cache
def _make_tc_math(n_edges):
    return pl.pallas_call(
        _tc_body,
        grid=(n_edges // B2,),
        in_specs=[
            pl.BlockSpec((B2, D), lambda i: (i, 0)),
            pl.BlockSpec((N_LABEL, D), lambda i: (0, 0)),
            pl.BlockSpec((N_LABEL, D), lambda i: (0, 0)),
            pl.BlockSpec((2, N_LABEL), lambda i: (0, 0)),
        ],
        out_specs=[
            pl.BlockSpec((1, B2), lambda i: (0, i)),
            pl.BlockSpec((1, B2), lambda i: (0, i)),
            pl.BlockSpec((2, B2), lambda i: (0, i)),
        ],
        out_shape=[
            jax.ShapeDtypeStruct((1, n_edges), jnp.float32),
            jax.ShapeDtypeStruct((1, n_edges), jnp.int32),
            jax.ShapeDtypeStruct((2, n_edges), jnp.int32),
        ],
    )


def kernel(h, node_edge, label_edge):
    le0, le1 = label_edge[0], label_edge[1]
    ne0, ne1 = node_edge[0], node_edge[1]
    parts = []
    rla = rlb = None
    for i, (sz, off) in enumerate(zip(SPLITS, OFFS)):
        s2_i, rla_i, rlb_i = _make_sc_gather(i == 0, sz, off)(
            h, ne0, ne1, le0, le1)
        if i == 0:
            rla, rlb = rla_i, rlb_i
        parts.append(s2_i)
    outs = [_make_tc_math(sz)(s2_i, rla, rlb, label_edge)
            for sz, s2_i in zip(SPLITS, parts)]
    n = node_edge.shape[1]
    return (jnp.concatenate([o[0] for o in outs], axis=1).reshape(n),
            jnp.concatenate([o[1] for o in outs], axis=1).reshape(n),
            jnp.concatenate([o[2] for o in outs], axis=1))


# speedup vs baseline: 1.1101x; 1.0092x over previous
import functools

import jax
import jax.numpy as jnp
from jax import lax
from jax.experimental import pallas as pl
from jax.experimental.pallas import tpu as pltpu
from jax.experimental.pallas import tpu_sc as plsc

N_NODES = 10000
N_EDGES = 320000
N_LABEL = 128
D = 128
DSUB = 8

NC = 2
NS = 16
NW = NC * NS
CHUNK = 200
SPLITS = (198400, 121600)
OFFS = (0, 198400)
GSPLIT = ((0, 104), (104, 96))


def _fire_gathers(h, idxa, idxb, bufa, bufb, sem):
    for off, ln in GSPLIT:
        pltpu.async_copy(h.at[idxa.at[pl.ds(off, ln)]],
                         bufa.at[pl.ds(off, ln)], sem)
        pltpu.async_copy(h.at[idxb.at[pl.ds(off, ln)]],
                         bufb.at[pl.ds(off, ln)], sem)


def _wait_gathers(h, idxa, idxb, bufa, bufb, sem):
    for off, ln in GSPLIT:
        pltpu.make_async_copy(h.at[idxa.at[pl.ds(off, ln)]],
                              bufa.at[pl.ds(off, ln)], sem).wait()
        pltpu.make_async_copy(h.at[idxb.at[pl.ds(off, ln)]],
                              bufb.at[pl.ds(off, ln)], sem).wait()


def _add_rows(bufa, bufb):
    def body(e2, carry):
        for r in range(2):
            e = e2 * 2 + r
            for c in range(DSUB):
                sl = pl.ds(c * 16, 16)
                bufa[e, sl] = bufa[e, sl] + bufb[e, sl]
        return carry
    lax.fori_loop(0, CHUNK // 2, body, 0)


def _sc_gather_body(do_label, n_edges, offset,
                    h, ne0, ne1, le0, le1, s2, rla, rlb,
                    idxa0, idxb0, idxa1, idxb1,
                    bufa0, bufb0, bufa1, bufb1, idxl,
                    gsem0, gsem1, wsem0, wsem1, lsem):
    epw = n_edges // NW
    nchunk = epw // CHUNK
    npair = (nchunk - 1) // 2
    c = lax.axis_index("c")
    s = lax.axis_index("s")
    wid = s * NC + c
    base0 = wid * epw

    if do_label:
        @pl.when(wid == 0)
        def _label():
            pltpu.sync_copy(le0, idxl)
            pltpu.async_copy(h.at[idxl], bufa0.at[pl.ds(0, N_LABEL)],
                             lsem).wait()
            pltpu.sync_copy(bufa0.at[pl.ds(0, N_LABEL)], rla)
            pltpu.sync_copy(le1, idxl)
            pltpu.async_copy(h.at[idxl], bufa0.at[pl.ds(0, N_LABEL)],
                             lsem).wait()
            pltpu.sync_copy(bufa0.at[pl.ds(0, N_LABEL)], rlb)

    def load_idx(j, idxa, idxb):
        base = pl.multiple_of(offset + base0 + j * CHUNK, 8)
        pltpu.sync_copy(ne0.at[pl.ds(base, CHUNK)], idxa)
        pltpu.sync_copy(ne1.at[pl.ds(base, CHUNK)], idxb)
        return base

    def process(j, idxa, idxb, bufa, bufb, gsem, wsem):
        _wait_gathers(h, idxa, idxb, bufa, bufb, gsem)
        _add_rows(bufa, bufb)
        b = pl.multiple_of(base0 + j * CHUNK, 8)
        pltpu.async_copy(bufa, s2.at[pl.ds(b, CHUNK)], wsem)

    load_idx(0, idxa0, idxb0)
    _fire_gathers(h, idxa0, idxb0, bufa0, bufb0, gsem0)

    def body(k, carry):
        j0 = 2 * k
        @pl.when(k > 0)
        def _():
            pltpu.make_async_copy(bufa1, s2.at[pl.ds(base0, CHUNK)],
                                  wsem1).wait()
        load_idx(j0 + 1, idxa1, idxb1)
        _fire_gathers(h, idxa1, idxb1, bufa1, bufb1, gsem1)
        process(j0, idxa0, idxb0, bufa0, bufb0, gsem0, wsem0)
        pltpu.make_async_copy(bufa0, s2.at[pl.ds(base0, CHUNK)],
                              wsem0).wait()
        load_idx(j0 + 2, idxa0, idxb0)
        _fire_gathers(h, idxa0, idxb0, bufa0, bufb0, gsem0)
        process(j0 + 1, idxa1, idxb1, bufa1, bufb1, gsem1, wsem1)
        return carry

    lax.fori_loop(0, npair, body, 0)

    process(nchunk - 1, idxa0, idxb0, bufa0, bufb0, gsem0, wsem0)
    pltpu.make_async_copy(bufa0, s2.at[pl.ds(base0, CHUNK)], wsem0).wait()
    pltpu.make_async_copy(bufa1, s2.at[pl.ds(base0, CHUNK)], wsem1).wait()


@functools.cache
def _make_sc_gather(do_label, n_edges, offset):
    return functools.partial(
        pl.kernel,
        out_type=[
            jax.ShapeDtypeStruct((n_edges, D), jnp.float32),
            jax.ShapeDtypeStruct((N_LABEL, D), jnp.float32),
            jax.ShapeDtypeStruct((N_LABEL, D), jnp.float32),
        ],
        mesh=plsc.VectorSubcoreMesh(core_axis_name="c", subcore_axis_name="s"),
        scratch_types=[
            pltpu.VMEM((CHUNK,), jnp.int32),
            pltpu.VMEM((CHUNK,), jnp.int32),
            pltpu.VMEM((CHUNK,), jnp.int32),
            pltpu.VMEM((CHUNK,), jnp.int32),
            pltpu.VMEM((CHUNK, D), jnp.float32),
            pltpu.VMEM((CHUNK, D), jnp.float32),
            pltpu.VMEM((CHUNK, D), jnp.float32),
            pltpu.VMEM((CHUNK, D), jnp.float32),
            pltpu.VMEM((N_LABEL,), jnp.int32),
            pltpu.SemaphoreType.DMA,
            pltpu.SemaphoreType.DMA,
            pltpu.SemaphoreType.DMA,
            pltpu.SemaphoreType.DMA,
            pltpu.SemaphoreType.DMA,
        ],
    )(functools.partial(_sc_gather_body, do_label, n_edges, offset))


B2 = 6400


def _tc_body(s2_ref, rla_ref, rlb_ref, le_ref,
             score_ref, idx_ref, pe_ref):
    en = s2_ref[...] * 0.5
    el = (rla_ref[...] + rlb_ref[...]) * 0.5
    sq_l = jnp.sum(el * el, axis=1, keepdims=True)
    dott = lax.dot_general(el, en, (((1,), (1,)), ((), ())),
                           preferred_element_type=jnp.float32)
    sq_nt = lax.dot_general(jnp.ones((1, D), jnp.float32), en * en,
                            (((1,), (1,)), ((), ())),
                            preferred_element_type=jnp.float32)
    d2t = jnp.maximum(sq_l + sq_nt - 2.0 * dott, 1e-12)
    mt = jnp.min(d2t, axis=0, keepdims=True)
    iot = lax.broadcasted_iota(jnp.int32, (N_LABEL, B2), 0)
    idxt = jnp.min(jnp.where(d2t == mt, iot, jnp.int32(N_LABEL)),
                   axis=0, keepdims=True)
    ohf = (iot == idxt).astype(jnp.float32)
    pe = lax.dot_general(le_ref[...].astype(jnp.float32), ohf,
                         (((1,), (0,)), ((), ())),
                         preferred_element_type=jnp.float32)
    score_ref[...] = -jnp.sqrt(mt)
    idx_ref[...] = idxt
    pe_ref[...] = pe.astype(jnp.int32)


@functools.cache
def _make_tc_math(n_edges):
    return pl.pallas_call(
        _tc_body,
        grid=(n_edges // B2,),
        in_specs=[
            pl.BlockSpec((B2, D), lambda i: (i, 0)),
            pl.BlockSpec((N_LABEL, D), lambda i: (0, 0)),
            pl.BlockSpec((N_LABEL, D), lambda i: (0, 0)),
            pl.BlockSpec((2, N_LABEL), lambda i: (0, 0)),
        ],
        out_specs=[
            pl.BlockSpec((1, B2), lambda i: (0, i)),
            pl.BlockSpec((1, B2), lambda i: (0, i)),
            pl.BlockSpec((2, B2), lambda i: (0, i)),
        ],
        out_shape=[
            jax.ShapeDtypeStruct((1, n_edges), jnp.float32),
            jax.ShapeDtypeStruct((1, n_edges), jnp.int32),
            jax.ShapeDtypeStruct((2, n_edges), jnp.int32),
        ],
    )


def kernel(h, node_edge, label_edge):
    le0, le1 = label_edge[0], label_edge[1]
    ne0, ne1 = node_edge[0], node_edge[1]
    parts = []
    rla = rlb = None
    for i, (sz, off) in enumerate(zip(SPLITS, OFFS)):
        s2_i, rla_i, rlb_i = _make_sc_gather(i == 0, sz, off)(
            h, ne0, ne1, le0, le1)
        if i == 0:
            rla, rlb = rla_i, rlb_i
        parts.append(s2_i)
    outs = [_make_tc_math(sz)(s2_i, rla, rlb, label_edge)
            for sz, s2_i in zip(SPLITS, parts)]
    n = node_edge.shape[1]
    return (jnp.concatenate([o[0] for o in outs], axis=1).reshape(n),
            jnp.concatenate([o[1] for o in outs], axis=1).reshape(n),
            jnp.concatenate([o[2] for o in outs], axis=1))
